# Initial kernel scaffold; baseline (speedup 1.0000x reference)
import jax
import jax.numpy as jnp
from jax.experimental import pallas as pl


def kernel(x, permutation):
    raise NotImplementedError("write your pallas kernel here")


# speedup vs baseline: 2.8625x; 2.8625x over previous
import functools

import jax
import jax.numpy as jnp
from jax import lax
from jax.experimental import pallas as pl
from jax.experimental.pallas import tpu as pltpu
from jax.experimental.pallas import tpu_sc as plsc

B, N, D = 4, 4096, 2048
ROWS = B * N

NC, NS = 2, 16
NW = NC * NS
RPW = ROWS // NW
CH = 16
NBUF = 2
NCHUNK = RPW // CH
L = 16


def _shuffle_sc(x2, perm):
    mesh = plsc.VectorSubcoreMesh(core_axis_name="c", subcore_axis_name="s")

    @functools.partial(
        pl.kernel,
        mesh=mesh,
        out_type=jax.ShapeDtypeStruct((ROWS, D), jnp.float32),
        scratch_types=[
            pltpu.VMEM((RPW,), jnp.int32),
            pltpu.VMEM((NBUF, CH, D), jnp.float32),
            pltpu.SemaphoreType.DMA,
            pltpu.SemaphoreType.DMA,
            pltpu.SemaphoreType.DMA,
            pltpu.SemaphoreType.DMA,
        ],
    )
    def k(x_hbm, perm_hbm, out_hbm, idx_v, buf, gsem0, gsem1, ssem0, ssem1):
        gsems = (gsem0, gsem1)
        ssems = (ssem0, ssem1)
        wid = lax.axis_index("s") * NC + lax.axis_index("c")
        base = wid * RPW
        b = base // N
        seg = base - b * N
        off = b * N

        pltpu.sync_copy(perm_hbm.at[pl.ds(seg, RPW)], idx_v)
        for j in range(RPW // L):
            idx_v[pl.ds(j * L, L)] = idx_v[pl.ds(j * L, L)] + off

        def gather(ci, slot):
            return pltpu.make_async_copy(
                x_hbm.at[idx_v.at[pl.ds(ci * CH, CH)]],
                buf.at[slot],
                gsems[slot],
            )

        def scatter(ci, slot):
            return pltpu.make_async_copy(
                buf.at[slot],
                out_hbm.at[pl.ds(base + ci * CH, CH)],
                ssems[slot],
            )

        for s in range(NBUF):
            gather(s, s).start()

        def step(i, carry):
            for s in range(NBUF):
                ci = i * NBUF + s
                gather(ci, s).wait()
                sc = scatter(ci, s)
                sc.start()
                sc.wait()
                nci = ci + NBUF

                @pl.when(nci < NCHUNK)
                def _():
                    gather(nci, s).start()

            return carry

        lax.fori_loop(0, NCHUNK // NBUF, step, 0)

    return k(x2, perm)


def kernel(x, permutation):
    x2 = x.reshape(ROWS, D)
    perm = permutation.astype(jnp.int32)
    out2 = _shuffle_sc(x2, perm)
    return out2.reshape(B, N, D)
